# baseline (speedup 1.0000x reference)
import functools

import jax
import jax.numpy as jnp
from jax.experimental import pallas as pl
from jax.experimental.pallas import tpu as pltpu

N = 134400
C = 64
G = 4
W = C // G
R = 4200
NB = N // R


def _body(stu_ref, tea_ref, siou_ref, tiou_ref, sgt_ref, tgt_ref, ms_ref,
          out_ref, acc_ref):
    pid = pl.program_id(0)

    @pl.when(pid == 0)
    def _init():
        acc_ref[0] = 0.0
        acc_ref[1] = 0.0

    mask = jnp.logical_and(tiou_ref[...] >= siou_ref[...],
                           tgt_ref[...] == sgt_ref[...])
    mask = jnp.logical_and(mask, ms_ref[...] != 0)
    mask_f = mask.astype(jnp.float32)

    t = tea_ref[...]
    s = stu_ref[...]

    gi = jax.lax.broadcasted_iota(jnp.int32, (C, C), 0) // W
    gj = jax.lax.broadcasted_iota(jnp.int32, (C, C), 1) // W
    p_mat = (gi == gj).astype(jnp.float32)

    et = jnp.exp(t)
    es = jnp.exp(s)
    bt = jax.lax.dot(et, p_mat, precision=jax.lax.Precision.HIGHEST)
    bs = jax.lax.dot(es, p_mat, precision=jax.lax.Precision.HIGHEST)

    elem = (et / bt) * (t - s) - jnp.log(bt / bs) * (1.0 / W)

    acc_ref[0] += jnp.sum(elem * mask_f)
    acc_ref[1] += jnp.sum(mask_f)

    @pl.when(pid == NB - 1)
    def _fin():
        out_ref[0, 0] = acc_ref[0] / (jnp.maximum(acc_ref[1], 1.0) * C)


@functools.partial(jax.jit, static_argnames=())
def kernel(stu_distri, tea_distri, stu_candidate_iou, tea_candidate_iou,
           stu_target_gt_idx, tea_target_gt_idx, Ms):
    siou = stu_candidate_iou.reshape(N, 1)
    tiou = tea_candidate_iou.reshape(N, 1)
    sgt = stu_target_gt_idx.astype(jnp.int32).reshape(N, 1)
    tgt = tea_target_gt_idx.astype(jnp.int32).reshape(N, 1)
    ms = Ms.astype(jnp.int32).reshape(N, 1)

    row_spec = pl.BlockSpec((R, C), lambda i: (i, 0))
    vec_spec = pl.BlockSpec((R, 1), lambda i: (i, 0))

    out = pl.pallas_call(
        _body,
        grid=(NB,),
        in_specs=[row_spec, row_spec, vec_spec, vec_spec, vec_spec, vec_spec,
                  vec_spec],
        out_specs=pl.BlockSpec(memory_space=pltpu.SMEM),
        out_shape=jax.ShapeDtypeStruct((1, 1), jnp.float32),
        scratch_shapes=[pltpu.SMEM((2,), jnp.float32)],
        compiler_params=pltpu.CompilerParams(
            dimension_semantics=("arbitrary",)),
    )(stu_distri, tea_distri, siou, tiou, sgt, tgt, ms)
    return out[0, 0]
